# Initial kernel scaffold; baseline (speedup 1.0000x reference)
import jax
import jax.numpy as jnp
from jax.experimental import pallas as pl


def kernel(x, edge_index, W0, b0, g0, be0, W1, b1, g1, be1, W2, b2):
    raise NotImplementedError("write your pallas kernel here")


# speedup vs baseline: 9.2955x; 9.2955x over previous
import functools

import jax
import jax.numpy as jnp
from jax import lax
from jax.experimental import pallas as pl
from jax.experimental.pallas import tpu as pltpu
from jax.experimental.pallas import tpu_sc as plsc

N = 10000
E = 320000
D = 128

NC = 2
NS = 16
NW = NC * NS
CW = 128
CHUNKS = 79
EPT = CHUNKS * CW
E_PAD = NW * EPT
N_PAD = 10112
RPT = N_PAD // NS
ZR = RPT // 8

@functools.cache
def _mesh():
    return plsc.VectorSubcoreMesh(core_axis_name="c", subcore_axis_name="s",
                                  num_cores=NC, num_subcores=NS)


def _deg_body(dst_hbm, out_hbm, dstv, hist):
    c = lax.axis_index("c")
    s = lax.axis_index("s")
    w = c * NS + s
    pltpu.sync_copy(dst_hbm.at[w], dstv)
    zero16 = jnp.zeros((16,), jnp.float32)

    def zb(i, _):
        hist[pl.ds(i * 16, 16)] = zero16
        return 0

    lax.fori_loop(0, N_PAD // 16, zb, 0)
    ones16 = jnp.ones((16,), jnp.float32)

    def acc_body(i, _):
        idx = dstv[i // 8, pl.ds((i % 8) * 16, 16)]
        plsc.addupdate_scatter(hist, [idx], ones16)
        return 0

    lax.fori_loop(0, CHUNKS * 8, acc_body, 0)
    pltpu.sync_copy(hist, out_hbm.at[w])


@functools.cache
def _deg_kernel():
    return pl.kernel(
        _deg_body,
        out_type=jax.ShapeDtypeStruct((NW, N_PAD), jnp.float32),
        mesh=_mesh(),
        scratch_types=[
            pltpu.VMEM((CHUNKS, CW), jnp.int32),
            pltpu.VMEM((N_PAD,), jnp.float32),
        ],
        compiler_params=pltpu.CompilerParams(needs_layout_passes=False),
    )


def _scatter_body(hp_hbm, src_hbm, dst_hbm, out_hbm, srcv, dstv, rows, zbuf,
                  acc_sh, gsem):
    c = lax.axis_index("c")
    s = lax.axis_index("s")
    w = c * NS + s
    pltpu.sync_copy(src_hbm.at[w], srcv)
    pltpu.sync_copy(dst_hbm.at[w], dstv)
    zero16 = jnp.zeros((16,), jnp.float32)

    def zb(i, _):
        zbuf[i // 8, pl.ds((i % 8) * 16, 16)] = zero16
        return 0

    lax.fori_loop(0, ZR * 8, zb, 0)
    for t in range(RPT // ZR):
        pltpu.sync_copy(zbuf, acc_sh.at[pl.ds(s * RPT + t * ZR, ZR)])
    plsc.subcore_barrier()

    def body(k, _):
        pltpu.async_copy(hp_hbm.at[srcv.at[k]], rows, gsem).wait()
        pltpu.sync_copy(rows, acc_sh.at[dstv.at[k]], add=True)
        return 0

    lax.fori_loop(0, CHUNKS, body, 0)
    plsc.subcore_barrier()
    pltpu.sync_copy(acc_sh.at[pl.ds(s * RPT, RPT)],
                    out_hbm.at[c, pl.ds(s * RPT, RPT)])


@functools.cache
def _scatter_kernel():
    return pl.kernel(
        _scatter_body,
        out_type=jax.ShapeDtypeStruct((NC, N_PAD, D), jnp.float32),
        mesh=_mesh(),
        scratch_types=[
            pltpu.VMEM((CHUNKS, CW), jnp.int32),
            pltpu.VMEM((CHUNKS, CW), jnp.int32),
            pltpu.VMEM((CW, D), jnp.float32),
            pltpu.VMEM((ZR, D), jnp.float32),
            pltpu.VMEM_SHARED((N_PAD, D), jnp.float32),
            pltpu.SemaphoreType.DMA,
        ],
    )


def _dinv_body(degp_ref, out_ref):
    deg = jnp.sum(degp_ref[...], axis=0, keepdims=True) + 1.0
    out_ref[...] = lax.rsqrt(deg)


def _pre_body(x_ref, w_ref, dinv_ref, hp_ref):
    hp_ref[...] = (
        jnp.dot(x_ref[...], w_ref[...], preferred_element_type=jnp.float32)
        * dinv_ref[...])


def _mid_body(acc_ref, hp_ref, dinv_ref, b_ref, g_ref, be_ref, w_ref, out_ref):
    dinv = dinv_ref[...]
    y = dinv * (acc_ref[0] + acc_ref[1] + hp_ref[...]) + b_ref[...]
    mask = (lax.broadcasted_iota(jnp.int32, (N_PAD, 1), 0) < N).astype(
        jnp.float32)
    ym = y * mask
    m = jnp.sum(ym, axis=0, keepdims=True) * (1.0 / N)
    v = jnp.sum(ym * ym, axis=0, keepdims=True) * (1.0 / N) - m * m
    r = (y - m) * lax.rsqrt(v + 1e-5) * g_ref[...] + be_ref[...]
    r = jnp.maximum(r, 0.0) * mask
    out_ref[...] = (
        jnp.dot(r, w_ref[...], preferred_element_type=jnp.float32) * dinv)


def _post_body(acc_ref, hp_ref, dinv_ref, b_ref, out_ref):
    y = dinv_ref[...] * (acc_ref[0] + acc_ref[1] + hp_ref[...]) + b_ref[...]
    r = jnp.maximum(y, 0.0)
    nrm = jnp.sqrt(jnp.sum(r * r, axis=1, keepdims=True))
    out_ref[...] = r / jnp.maximum(nrm, 1e-12)


def _tc(body, out_shape):
    return pl.pallas_call(body, out_shape=jax.ShapeDtypeStruct(out_shape,
                                                               jnp.float32))


def kernel(x, edge_index, W0, b0, g0, be0, W1, b1, g1, be1, W2, b2):
    src = jnp.concatenate(
        [edge_index[0], jnp.zeros((E_PAD - E,), jnp.int32)]).reshape(
            NW, CHUNKS, CW)
    dst = jnp.concatenate(
        [edge_index[1],
         jnp.full((E_PAD - E,), N_PAD - 1, jnp.int32)]).reshape(
             NW, CHUNKS, CW)
    xp = jnp.pad(x, ((0, N_PAD - N), (0, 0)))

    deg_parts = _deg_kernel()(dst)
    dinv = _tc(_dinv_body, (1, N_PAD))(deg_parts).reshape(N_PAD, 1)

    hp = _tc(_pre_body, (N_PAD, D))(xp, W0, dinv)
    acc = _scatter_kernel()(hp, src, dst)
    hp = _tc(_mid_body, (N_PAD, D))(acc, hp, dinv, b0.reshape(1, D),
                                    g0.reshape(1, D), be0.reshape(1, D), W1)
    acc = _scatter_kernel()(hp, src, dst)
    hp = _tc(_mid_body, (N_PAD, D))(acc, hp, dinv, b1.reshape(1, D),
                                    g1.reshape(1, D), be1.reshape(1, D), W2)
    acc = _scatter_kernel()(hp, src, dst)
    emb = _tc(_post_body, (N_PAD, D))(acc, hp, dinv, b2.reshape(1, D))
    return emb[:N]


# speedup vs baseline: 9.8628x; 1.0610x over previous
import functools

import jax
import jax.numpy as jnp
from jax import lax
from jax.experimental import pallas as pl
from jax.experimental.pallas import tpu as pltpu
from jax.experimental.pallas import tpu_sc as plsc

N = 10000
E = 320000
D = 128

NC = 2
NS = 16
NW = NC * NS
CW = 64
CHUNKS = 158
EPT = CHUNKS * CW
E_PAD = NW * EPT
N_PAD = 10112
RPT = N_PAD // NS
ZR = 8
SHIFT = 14
SMASK = (1 << SHIFT) - 1


@functools.cache
def _mesh():
    return plsc.VectorSubcoreMesh(core_axis_name="c", subcore_axis_name="s",
                                  num_cores=NC, num_subcores=NS)


def _deg_body(edges_hbm, out_hbm, ev, hist):
    c = lax.axis_index("c")
    s = lax.axis_index("s")
    w = c * NS + s
    pltpu.sync_copy(edges_hbm.at[w], ev)
    zero16 = jnp.zeros((16,), jnp.float32)

    def zb(i, _):
        hist[pl.ds(i * 16, 16)] = zero16
        return 0

    lax.fori_loop(0, N_PAD // 16, zb, 0)
    ones16 = jnp.ones((16,), jnp.float32)
    nvec = CW // 16

    def acc_body(i, _):
        idx = lax.shift_right_logical(
            ev[i // nvec, pl.ds((i % nvec) * 16, 16)], SHIFT)
        plsc.addupdate_scatter(hist, [idx], ones16)
        return 0

    lax.fori_loop(0, CHUNKS * nvec, acc_body, 0)
    pltpu.sync_copy(hist, out_hbm.at[w])


@functools.cache
def _deg_kernel():
    return pl.kernel(
        _deg_body,
        out_type=jax.ShapeDtypeStruct((NW, N_PAD), jnp.float32),
        mesh=_mesh(),
        scratch_types=[
            pltpu.VMEM((CHUNKS, CW), jnp.int32),
            pltpu.VMEM((N_PAD,), jnp.float32),
        ],
        compiler_params=pltpu.CompilerParams(needs_layout_passes=False),
    )


def _scatter_body(hp_hbm, edges_hbm, out_hbm, ev, sidx, didx, rows, zbuf,
                  acc_sh, gsem0, gsem1):
    c = lax.axis_index("c")
    s = lax.axis_index("s")
    w = c * NS + s
    pltpu.sync_copy(edges_hbm.at[w], ev)
    zero16 = jnp.zeros((16,), jnp.float32)

    def zb(i, _):
        zbuf[i // 8, pl.ds((i % 8) * 16, 16)] = zero16
        return 0

    lax.fori_loop(0, ZR * 8, zb, 0)
    for t in range(RPT // ZR):
        pltpu.sync_copy(zbuf, acc_sh.at[pl.ds(s * RPT + t * ZR, ZR)])
    plsc.subcore_barrier()

    def unpack(k, buf):
        for t in range(CW // 16):
            v = ev[k, pl.ds(t * 16, 16)]
            sidx[buf, pl.ds(t * 16, 16)] = v & SMASK
            didx[buf, pl.ds(t * 16, 16)] = lax.shift_right_logical(v, SHIFT)

    def gather(buf, sem):
        return pltpu.async_copy(hp_hbm.at[sidx.at[buf]], rows.at[buf], sem)

    def gwait(buf, sem):
        pltpu.make_async_copy(hp_hbm.at[sidx.at[buf]], rows.at[buf],
                              sem).wait()

    def scatter(buf):
        pltpu.sync_copy(rows.at[buf], acc_sh.at[didx.at[buf]], add=True)

    unpack(0, 0)
    gather(0, gsem0)

    def body(j, _):
        k = 2 * j
        unpack(k + 1, 1)
        gather(1, gsem1)
        gwait(0, gsem0)
        scatter(0)

        @pl.when(k + 2 < CHUNKS)
        def _():
            unpack(k + 2, 0)
            gather(0, gsem0)

        gwait(1, gsem1)
        scatter(1)
        return 0

    lax.fori_loop(0, CHUNKS // 2, body, 0)
    plsc.subcore_barrier()
    pltpu.sync_copy(acc_sh.at[pl.ds(s * RPT, RPT)],
                    out_hbm.at[c, pl.ds(s * RPT, RPT)])


@functools.cache
def _scatter_kernel():
    return pl.kernel(
        _scatter_body,
        out_type=jax.ShapeDtypeStruct((NC, N_PAD, D), jnp.float32),
        mesh=_mesh(),
        scratch_types=[
            pltpu.VMEM((CHUNKS, CW), jnp.int32),
            pltpu.VMEM((2, CW), jnp.int32),
            pltpu.VMEM((2, CW), jnp.int32),
            pltpu.VMEM((2, CW, D), jnp.float32),
            pltpu.VMEM((ZR, D), jnp.float32),
            pltpu.VMEM_SHARED((N_PAD, D), jnp.float32),
            pltpu.SemaphoreType.DMA,
            pltpu.SemaphoreType.DMA,
        ],
    )


def _dinv_body(degp_ref, out_ref):
    deg = jnp.sum(degp_ref[...], axis=0, keepdims=True) + 1.0
    out_ref[...] = lax.rsqrt(deg)


def _pre_body(x_ref, w_ref, dinv_ref, hp_ref):
    hp_ref[...] = (
        jnp.dot(x_ref[...], w_ref[...], preferred_element_type=jnp.float32)
        * dinv_ref[...])


def _mid_body(acc_ref, hp_ref, dinv_ref, b_ref, g_ref, be_ref, w_ref, out_ref):
    dinv = dinv_ref[...]
    y = dinv * (acc_ref[0] + acc_ref[1] + hp_ref[...]) + b_ref[...]
    mask = (lax.broadcasted_iota(jnp.int32, (N_PAD, 1), 0) < N).astype(
        jnp.float32)
    ym = y * mask
    m = jnp.sum(ym, axis=0, keepdims=True) * (1.0 / N)
    v = jnp.sum(ym * ym, axis=0, keepdims=True) * (1.0 / N) - m * m
    r = (y - m) * lax.rsqrt(v + 1e-5) * g_ref[...] + be_ref[...]
    r = jnp.maximum(r, 0.0) * mask
    out_ref[...] = (
        jnp.dot(r, w_ref[...], preferred_element_type=jnp.float32) * dinv)


def _post_body(acc_ref, hp_ref, dinv_ref, b_ref, out_ref):
    y = dinv_ref[...] * (acc_ref[0] + acc_ref[1] + hp_ref[...]) + b_ref[...]
    r = jnp.maximum(y, 0.0)
    nrm = jnp.sqrt(jnp.sum(r * r, axis=1, keepdims=True))
    out_ref[...] = r / jnp.maximum(nrm, 1e-12)


def _tc(body, out_shape):
    return pl.pallas_call(body, out_shape=jax.ShapeDtypeStruct(out_shape,
                                                               jnp.float32))


def kernel(x, edge_index, W0, b0, g0, be0, W1, b1, g1, be1, W2, b2):
    packed = edge_index[0] | (edge_index[1] << SHIFT)
    edges = jnp.concatenate(
        [packed, jnp.full((E_PAD - E,), (N_PAD - 1) << SHIFT,
                          jnp.int32)]).reshape(NW, CHUNKS, CW)
    xp = jnp.pad(x, ((0, N_PAD - N), (0, 0)))

    deg_parts = _deg_kernel()(edges)
    dinv = _tc(_dinv_body, (1, N_PAD))(deg_parts).reshape(N_PAD, 1)

    hp = _tc(_pre_body, (N_PAD, D))(xp, W0, dinv)
    acc = _scatter_kernel()(hp, edges)
    hp = _tc(_mid_body, (N_PAD, D))(acc, hp, dinv, b0.reshape(1, D),
                                    g0.reshape(1, D), be0.reshape(1, D), W1)
    acc = _scatter_kernel()(hp, edges)
    hp = _tc(_mid_body, (N_PAD, D))(acc, hp, dinv, b1.reshape(1, D),
                                    g1.reshape(1, D), be1.reshape(1, D), W2)
    acc = _scatter_kernel()(hp, edges)
    emb = _tc(_post_body, (N_PAD, D))(acc, hp, dinv, b2.reshape(1, D))
    return emb[:N]


# speedup vs baseline: 10.4896x; 1.0636x over previous
import functools

import jax
import jax.numpy as jnp
from jax import lax
from jax.experimental import pallas as pl
from jax.experimental.pallas import tpu as pltpu
from jax.experimental.pallas import tpu_sc as plsc

N = 10000
E = 320000
D = 128

NC = 2
NS = 16
NW = NC * NS
CW = 64
CHUNKS = 158
EPT = CHUNKS * CW
E_PAD = NW * EPT
F0 = 158
F1 = 158
TOTC = NS * (F0 + F1)
FMAX = max(F0, F1)
N_PAD = 10112
RPT = N_PAD // NS
ZR = 8
SHIFT = 14
SMASK = (1 << SHIFT) - 1


@functools.cache
def _mesh():
    return plsc.VectorSubcoreMesh(core_axis_name="c", subcore_axis_name="s",
                                  num_cores=NC, num_subcores=NS)


def _deg_body(edges_hbm, out_hbm, ev, hist):
    c = lax.axis_index("c")
    s = lax.axis_index("s")
    w = c * NS + s
    pltpu.sync_copy(edges_hbm.at[w], ev)
    zero16 = jnp.zeros((16,), jnp.float32)

    def zb(i, _):
        hist[pl.ds(i * 16, 16)] = zero16
        return 0

    lax.fori_loop(0, N_PAD // 16, zb, 0)
    ones16 = jnp.ones((16,), jnp.float32)
    nvec = CW // 16

    def acc_body(i, _):
        idx = lax.shift_right_logical(
            ev[i // nvec, pl.ds((i % nvec) * 16, 16)], SHIFT)
        plsc.addupdate_scatter(hist, [idx], ones16)
        return 0

    lax.fori_loop(0, CHUNKS * nvec, acc_body, 0)
    pltpu.sync_copy(hist, out_hbm.at[w])


@functools.cache
def _deg_kernel():
    return pl.kernel(
        _deg_body,
        out_type=jax.ShapeDtypeStruct((NW, N_PAD), jnp.float32),
        mesh=_mesh(),
        scratch_types=[
            pltpu.VMEM((CHUNKS, CW), jnp.int32),
            pltpu.VMEM((N_PAD,), jnp.float32),
        ],
        compiler_params=pltpu.CompilerParams(needs_layout_passes=False),
    )


def _scatter_body(hp_hbm, edges_hbm, out_hbm, ev, sidx, didx, rows, zbuf,
                  acc_sh, gsem0, gsem1):
    c = lax.axis_index("c")
    s = lax.axis_index("s")
    w = c * NS + s
    pltpu.sync_copy(edges_hbm.at[w], ev)
    zero16 = jnp.zeros((16,), jnp.float32)

    def zb(i, _):
        zbuf[i // 8, pl.ds((i % 8) * 16, 16)] = zero16
        return 0

    lax.fori_loop(0, ZR * 8, zb, 0)
    for t in range(RPT // ZR):
        pltpu.sync_copy(zbuf, acc_sh.at[pl.ds(s * RPT + t * ZR, ZR)])
    plsc.subcore_barrier()

    def unpack(k, buf):
        for t in range(CW // 16):
            v = ev[k, pl.ds(t * 16, 16)]
            sidx[buf, pl.ds(t * 16, 16)] = v & SMASK
            didx[buf, pl.ds(t * 16, 16)] = lax.shift_right_logical(v, SHIFT)

    def gather(buf, sem):
        return pltpu.async_copy(hp_hbm.at[sidx.at[buf]], rows.at[buf], sem)

    def gwait(buf, sem):
        pltpu.make_async_copy(hp_hbm.at[sidx.at[buf]], rows.at[buf],
                              sem).wait()

    def scatter(buf):
        pltpu.sync_copy(rows.at[buf], acc_sh.at[didx.at[buf]], add=True)

    def chunk_loop(nchunks):
        unpack(0, 0)
        gather(0, gsem0)

        def body(j, _):
            k = 2 * j
            unpack(k + 1, 1)
            gather(1, gsem1)
            gwait(0, gsem0)
            scatter(0)
            unpack(jnp.minimum(k + 2, nchunks - 1), 0)
            gather(0, gsem0)
            gwait(1, gsem1)
            scatter(1)
            return 0

        lax.fori_loop(0, nchunks // 2, body, 0)
        gwait(0, gsem0)

    @pl.when(c == 0)
    def _():
        chunk_loop(F0)

    @pl.when(c == 1)
    def _():
        chunk_loop(F1)

    plsc.subcore_barrier()
    pltpu.sync_copy(acc_sh.at[pl.ds(s * RPT, RPT)],
                    out_hbm.at[c, pl.ds(s * RPT, RPT)])


@functools.cache
def _scatter_kernel():
    return pl.kernel(
        _scatter_body,
        out_type=jax.ShapeDtypeStruct((NC, N_PAD, D), jnp.float32),
        mesh=_mesh(),
        scratch_types=[
            pltpu.VMEM((FMAX, CW), jnp.int32),
            pltpu.VMEM((2, CW), jnp.int32),
            pltpu.VMEM((2, CW), jnp.int32),
            pltpu.VMEM((2, CW, D), jnp.float32),
            pltpu.VMEM((ZR, D), jnp.float32),
            pltpu.VMEM_SHARED((N_PAD, D), jnp.float32),
            pltpu.SemaphoreType.DMA,
            pltpu.SemaphoreType.DMA,
        ],
    )


def _dinv_body(degp_ref, out_ref):
    deg = jnp.sum(degp_ref[...], axis=0, keepdims=True) + 1.0
    out_ref[...] = lax.rsqrt(deg)


def _pre_body(x_ref, w_ref, dinv_ref, hp_ref):
    hp_ref[...] = (
        jnp.dot(x_ref[...], w_ref[...], preferred_element_type=jnp.float32)
        * dinv_ref[...])


def _mid_body(acc_ref, hp_ref, dinv_ref, b_ref, g_ref, be_ref, w_ref, out_ref):
    dinv = dinv_ref[...]
    y = dinv * (acc_ref[0] + acc_ref[1] + hp_ref[...]) + b_ref[...]
    mask = (lax.broadcasted_iota(jnp.int32, (N_PAD, 1), 0) < N).astype(
        jnp.float32)
    ym = y * mask
    m = jnp.sum(ym, axis=0, keepdims=True) * (1.0 / N)
    v = jnp.sum(ym * ym, axis=0, keepdims=True) * (1.0 / N) - m * m
    r = (y - m) * lax.rsqrt(v + 1e-5) * g_ref[...] + be_ref[...]
    r = jnp.maximum(r, 0.0) * mask
    out_ref[...] = (
        jnp.dot(r, w_ref[...], preferred_element_type=jnp.float32) * dinv)


def _post_body(acc_ref, hp_ref, dinv_ref, b_ref, out_ref):
    y = dinv_ref[...] * (acc_ref[0] + acc_ref[1] + hp_ref[...]) + b_ref[...]
    r = jnp.maximum(y, 0.0)
    nrm = jnp.sqrt(jnp.sum(r * r, axis=1, keepdims=True))
    out_ref[...] = r / jnp.maximum(nrm, 1e-12)


def _tc(body, out_shape):
    return pl.pallas_call(body, out_shape=jax.ShapeDtypeStruct(out_shape,
                                                               jnp.float32))


def kernel(x, edge_index, W0, b0, g0, be0, W1, b1, g1, be1, W2, b2):
    packed = edge_index[0] | (edge_index[1] << SHIFT)
    pad = jnp.full((E_PAD - E,), (N_PAD - 1) << SHIFT, jnp.int32)
    edges = jnp.concatenate([packed, pad]).reshape(NW, CHUNKS, CW)
    pad2 = jnp.full((TOTC * CW - E,), (N_PAD - 1) << SHIFT, jnp.int32)
    flat = jnp.concatenate([packed, pad2]).reshape(TOTC, CW)
    part0 = flat[:NS * F0].reshape(NS, F0, CW)
    part1 = jnp.concatenate(
        [flat[NS * F0:].reshape(NS, F1, CW),
         jnp.full((NS, FMAX - F1, CW), (N_PAD - 1) << SHIFT, jnp.int32)],
        axis=1)
    edges_flat = jnp.concatenate(
        [part0, jnp.zeros((NS, FMAX - F0, CW), jnp.int32)], axis=1
    ) if FMAX > F0 else part0
    edges_flat = jnp.concatenate([edges_flat, part1], axis=0)
    xp = jnp.pad(x, ((0, N_PAD - N), (0, 0)))

    deg_parts = _deg_kernel()(edges)
    dinv = _tc(_dinv_body, (1, N_PAD))(deg_parts).reshape(N_PAD, 1)

    hp = _tc(_pre_body, (N_PAD, D))(xp, W0, dinv)
    acc = _scatter_kernel()(hp, edges_flat)
    hp = _tc(_mid_body, (N_PAD, D))(acc, hp, dinv, b0.reshape(1, D),
                                    g0.reshape(1, D), be0.reshape(1, D), W1)
    acc = _scatter_kernel()(hp, edges_flat)
    hp = _tc(_mid_body, (N_PAD, D))(acc, hp, dinv, b1.reshape(1, D),
                                    g1.reshape(1, D), be1.reshape(1, D), W2)
    acc = _scatter_kernel()(hp, edges_flat)
    emb = _tc(_post_body, (N_PAD, D))(acc, hp, dinv, b2.reshape(1, D))
    return emb[:N]
